# baseline (speedup 1.0000x reference)
import functools

import jax
import jax.numpy as jnp
from jax import lax
from jax.experimental import pallas as pl
from jax.experimental.pallas import tpu as pltpu
from jax.experimental.pallas import tpu_sc as plsc

_TCHUNK = 20480
_CHD = 1152
_VSC = 184320
_LN2 = 0.6931471805599453


def _tc_body(logits_ref, noise_ref, idx_ref, logp_ref, cnt_ref, m_ref,
             best_score, best_idx, best_logp, cnt_acc, *, vbase, vlim):
    step = pl.program_id(0)
    nsteps = pl.num_programs(0)

    @pl.when(step == 0)
    def _init():
        best_score[...] = jnp.full(best_score.shape, -1.0, best_score.dtype)
        best_idx[...] = jnp.zeros(best_idx.shape, best_idx.dtype)
        best_logp[...] = jnp.ones(best_logp.shape, best_logp.dtype)
        cnt_acc[...] = jnp.zeros(cnt_acc.shape, cnt_acc.dtype)

    x = logits_ref[...]
    u = noise_ref[...]
    probs = jax.nn.sigmoid(x) * 0.999 + 0.0005
    w = -jnp.log(u)
    ratio = probs / w
    col = (lax.broadcasted_iota(jnp.int32, x.shape, 1)
           + (vbase + step * x.shape[1]))
    valid = col < vlim
    ratio = jnp.where(valid, ratio, -1.0)
    gate = jnp.where(valid & (u < probs), 1.0, 0.0)

    m = jnp.max(ratio, axis=1, keepdims=True)
    hit = ratio == m
    local_idx = jnp.min(jnp.where(hit, col, vlim), axis=1, keepdims=True)
    p_at = jnp.max(jnp.where(col == local_idx, probs, -1.0),
                   axis=1, keepdims=True)
    local_cnt = jnp.sum(gate, axis=1, keepdims=True)

    better = m > best_score[...]
    best_idx[...] = jnp.where(better, local_idx, best_idx[...])
    best_logp[...] = jnp.where(better, p_at, best_logp[...])
    best_score[...] = jnp.where(better, m, best_score[...])
    cnt_acc[...] = cnt_acc[...] + local_cnt

    @pl.when(step == nsteps - 1)
    def _fin():
        idx_ref[...] = best_idx[...]
        logp_ref[...] = jnp.log(best_logp[...])
        cnt_ref[...] = cnt_acc[...]
        m_ref[...] = best_score[...]


def _tc_call(logits, noise, vbase, vlim):
    B = logits.shape[0]
    first = vbase // _TCHUNK
    grid = ((vlim - vbase + _TCHUNK - 1) // _TCHUNK,)
    out_shape = [
        jax.ShapeDtypeStruct((B, 1), jnp.int32),
        jax.ShapeDtypeStruct((B, 1), jnp.float32),
        jax.ShapeDtypeStruct((B, 1), jnp.float32),
        jax.ShapeDtypeStruct((B, 1), jnp.float32),
    ]
    return pl.pallas_call(
        functools.partial(_tc_body, vbase=vbase, vlim=vlim),
        grid=grid,
        in_specs=[
            pl.BlockSpec((B, _TCHUNK), lambda i: (0, i + first)),
            pl.BlockSpec((B, _TCHUNK), lambda i: (0, i + first)),
        ],
        out_specs=[pl.BlockSpec((B, 1), lambda i: (0, 0))] * 4,
        out_shape=out_shape,
        scratch_shapes=[
            pltpu.VMEM((B, 1), jnp.float32),
            pltpu.VMEM((B, 1), jnp.int32),
            pltpu.VMEM((B, 1), jnp.float32),
            pltpu.VMEM((B, 1), jnp.float32),
        ],
        compiler_params=pltpu.CompilerParams(
            dimension_semantics=("arbitrary",)),
    )(logits, noise)


def _neg_ln(u):
    bits = lax.bitcast_convert_type(u, jnp.int32)
    e = (bits >> 23) - 127
    mant = lax.bitcast_convert_type(
        (bits & jnp.int32(0x007FFFFF)) | jnp.int32(0x3F800000), jnp.float32)
    near1 = u > 0.75
    num = jnp.where(near1, u - 1.0, mant - 1.0)
    den = jnp.where(near1, u + 1.0, mant + 1.0)
    y = num / den
    y2 = y * y
    h = 1.0 / 9.0
    h = h * y2 + 1.0 / 7.0
    h = h * y2 + 0.2
    h = h * y2 + 1.0 / 3.0
    h = h * y2 + 1.0
    s2 = 2.0 * (y * h)
    base = jnp.where(near1, 0.0, e.astype(jnp.float32) * _LN2)
    return -(base + s2)


def _sc_call(logits, noise, vstart, vend):
    B, V = logits.shape
    nw = 32
    segw = (vend - vstart) // nw
    nch = segw // _CHD
    mesh = plsc.VectorSubcoreMesh(core_axis_name="c", subcore_axis_name="s")
    out_type = [
        jax.ShapeDtypeStruct((nw, B, 16), jnp.float32),
        jax.ShapeDtypeStruct((nw, B, 16), jnp.int32),
        jax.ShapeDtypeStruct((nw, B, 16), jnp.float32),
        jax.ShapeDtypeStruct((nw, B, 16), jnp.float32),
    ]

    @functools.partial(
        pl.kernel, mesh=mesh, out_type=out_type,
        scratch_types=[
            pltpu.VMEM((B, _CHD), jnp.float32),
            pltpu.VMEM((B, _CHD), jnp.float32),
            pltpu.VMEM((B, 16), jnp.float32),
            pltpu.VMEM((B, 16), jnp.int32),
            pltpu.VMEM((B, 16), jnp.float32),
            pltpu.VMEM((B, 16), jnp.float32),
        ],
    )
    def sck(logits_hbm, noise_hbm, om_hbm, oi_hbm, ot_hbm, oc_hbm,
            xbuf, ubuf, mslab, islab, tslab, cslab):
        w = lax.axis_index("s") * 2 + lax.axis_index("c")
        lanes = lax.iota(jnp.int32, 16)
        colbase = vstart + w * segw

        mslab[...] = jnp.full((B, 16), -1.0, jnp.float32)
        islab[...] = jnp.zeros((B, 16), jnp.int32)
        tslab[...] = jnp.ones((B, 16), jnp.float32)
        cslab[...] = jnp.zeros((B, 16), jnp.float32)

        def do_block(start, nvec):
            def row(r, _):
                def vec(j, carry2):
                    acc_m, acc_i, acc_t, acc_c = carry2
                    x = xbuf[r, pl.ds(j * 16, 16)]
                    u = ubuf[r, pl.ds(j * 16, 16)]
                    t = jnp.exp(-x)
                    a = 0.0005 * t + 0.9995
                    onep = 1.0 + t
                    wv = _neg_ln(u)
                    rr = a / (onep * wv)
                    gate = u * onep < a
                    acc_c = acc_c + jnp.where(gate, 1.0, 0.0)
                    better = rr > acc_m
                    pos = (start + j * 16) + lanes
                    acc_m = jnp.where(better, rr, acc_m)
                    acc_i = jnp.where(better, pos, acc_i)
                    acc_t = jnp.where(better, t, acc_t)
                    return acc_m, acc_i, acc_t, acc_c

                carry = (mslab[r], islab[r], tslab[r], cslab[r])
                acc_m, acc_i, acc_t, acc_c = lax.fori_loop(
                    0, nvec, vec, carry, unroll=4)
                mslab[r] = acc_m
                islab[r] = acc_i
                tslab[r] = acc_t
                cslab[r] = acc_c
                return 0

            lax.fori_loop(0, B, row, 0)

        def chunk(c, _):
            start = colbase + c * _CHD
            pltpu.sync_copy(logits_hbm.at[:, pl.ds(start, _CHD)], xbuf)
            pltpu.sync_copy(noise_hbm.at[:, pl.ds(start, _CHD)], ubuf)
            do_block(start, _CHD // 16)
            return 0

        lax.fori_loop(0, nch, chunk, 0)

        pltpu.sync_copy(mslab, om_hbm.at[w])
        pltpu.sync_copy(islab, oi_hbm.at[w])
        pltpu.sync_copy(tslab, ot_hbm.at[w])
        pltpu.sync_copy(cslab, oc_hbm.at[w])

    return sck(logits, noise)


def kernel(logits, noise):
    B, V = logits.shape
    idx_tc, logp_tc, cnt_tc, m_tc = _tc_call(logits, noise, _VSC, V)
    sc_m, sc_i, sc_t, sc_c = _sc_call(logits, noise, 0, _VSC)

    sc_m2 = jnp.swapaxes(sc_m, 0, 1).reshape(B, -1)
    sc_i2 = jnp.swapaxes(sc_i, 0, 1).reshape(B, -1)
    sc_t2 = jnp.swapaxes(sc_t, 0, 1).reshape(B, -1)
    sc_mrow = jnp.max(sc_m2, axis=1)
    hit = sc_m2 == sc_mrow[:, None]
    sc_idx = jnp.min(jnp.where(hit, sc_i2, V), axis=1)
    sc_tat = jnp.max(jnp.where(hit, sc_t2, -1.0), axis=1)
    sc_logp = jnp.log(0.999 / (1.0 + sc_tat) + 0.0005)
    sc_cnt = jnp.sum(sc_c, axis=(0, 2))

    better = sc_mrow >= m_tc[:, 0]
    idx = jnp.where(better, sc_idx, idx_tc[:, 0])
    chosen_logp = jnp.where(better, sc_logp, logp_tc[:, 0])
    gate_count = cnt_tc[:, 0] + sc_cnt
    return (idx, chosen_logp, gate_count)


# speedup vs baseline: 1.0430x; 1.0430x over previous
import functools

import jax
import jax.numpy as jnp
from jax import lax
from jax.experimental import pallas as pl
from jax.experimental.pallas import tpu as pltpu
from jax.experimental.pallas import tpu_sc as plsc

_TCHUNK = 20480
_CHD = 1280
_VSC = 163840
_LN2 = 0.6931471805599453


def _tc_body(logits_ref, noise_ref, idx_ref, logp_ref, cnt_ref, m_ref,
             best_score, best_idx, best_logp, cnt_acc, *, vbase, vlim):
    step = pl.program_id(0)
    nsteps = pl.num_programs(0)

    @pl.when(step == 0)
    def _init():
        best_score[...] = jnp.full(best_score.shape, -1.0, best_score.dtype)
        best_idx[...] = jnp.zeros(best_idx.shape, best_idx.dtype)
        best_logp[...] = jnp.ones(best_logp.shape, best_logp.dtype)
        cnt_acc[...] = jnp.zeros(cnt_acc.shape, cnt_acc.dtype)

    x = logits_ref[...]
    u = noise_ref[...]
    probs = jax.nn.sigmoid(x) * 0.999 + 0.0005
    w = -jnp.log(u)
    ratio = probs / w
    col = (lax.broadcasted_iota(jnp.int32, x.shape, 1)
           + (vbase + step * x.shape[1]))
    valid = col < vlim
    ratio = jnp.where(valid, ratio, -1.0)
    gate = jnp.where(valid & (u < probs), 1.0, 0.0)

    m = jnp.max(ratio, axis=1, keepdims=True)
    hit = ratio == m
    local_idx = jnp.min(jnp.where(hit, col, vlim), axis=1, keepdims=True)
    p_at = jnp.max(jnp.where(col == local_idx, probs, -1.0),
                   axis=1, keepdims=True)
    local_cnt = jnp.sum(gate, axis=1, keepdims=True)

    better = m > best_score[...]
    best_idx[...] = jnp.where(better, local_idx, best_idx[...])
    best_logp[...] = jnp.where(better, p_at, best_logp[...])
    best_score[...] = jnp.where(better, m, best_score[...])
    cnt_acc[...] = cnt_acc[...] + local_cnt

    @pl.when(step == nsteps - 1)
    def _fin():
        idx_ref[...] = best_idx[...]
        logp_ref[...] = jnp.log(best_logp[...])
        cnt_ref[...] = cnt_acc[...]
        m_ref[...] = best_score[...]


def _tc_call(logits, noise, vbase, vlim):
    B = logits.shape[0]
    first = vbase // _TCHUNK
    grid = ((vlim - vbase + _TCHUNK - 1) // _TCHUNK,)
    out_shape = [
        jax.ShapeDtypeStruct((B, 1), jnp.int32),
        jax.ShapeDtypeStruct((B, 1), jnp.float32),
        jax.ShapeDtypeStruct((B, 1), jnp.float32),
        jax.ShapeDtypeStruct((B, 1), jnp.float32),
    ]
    return pl.pallas_call(
        functools.partial(_tc_body, vbase=vbase, vlim=vlim),
        grid=grid,
        in_specs=[
            pl.BlockSpec((B, _TCHUNK), lambda i: (0, i + first)),
            pl.BlockSpec((B, _TCHUNK), lambda i: (0, i + first)),
        ],
        out_specs=[pl.BlockSpec((B, 1), lambda i: (0, 0))] * 4,
        out_shape=out_shape,
        scratch_shapes=[
            pltpu.VMEM((B, 1), jnp.float32),
            pltpu.VMEM((B, 1), jnp.int32),
            pltpu.VMEM((B, 1), jnp.float32),
            pltpu.VMEM((B, 1), jnp.float32),
        ],
        compiler_params=pltpu.CompilerParams(
            dimension_semantics=("arbitrary",)),
    )(logits, noise)


def _neg_ln(u):
    bits = lax.bitcast_convert_type(u, jnp.int32)
    e = (bits >> 23) - 127
    mant = lax.bitcast_convert_type(
        (bits & jnp.int32(0x007FFFFF)) | jnp.int32(0x3F800000), jnp.float32)
    near1 = u > 0.75
    num = jnp.where(near1, u - 1.0, mant - 1.0)
    den = jnp.where(near1, u + 1.0, mant + 1.0)
    y = num / den
    y2 = y * y
    h = 1.0 / 9.0
    h = h * y2 + 1.0 / 7.0
    h = h * y2 + 0.2
    h = h * y2 + 1.0 / 3.0
    h = h * y2 + 1.0
    s2 = 2.0 * (y * h)
    base = jnp.where(near1, 0.0, e.astype(jnp.float32) * _LN2)
    return -(base + s2)


def _sc_call(logits, noise, vstart, vend):
    B, V = logits.shape
    nw = 32
    segw = (vend - vstart) // nw
    nch = segw // _CHD
    mesh = plsc.VectorSubcoreMesh(core_axis_name="c", subcore_axis_name="s")
    out_type = [
        jax.ShapeDtypeStruct((nw, B, 16), jnp.float32),
        jax.ShapeDtypeStruct((nw, B, 16), jnp.int32),
        jax.ShapeDtypeStruct((nw, B, 16), jnp.float32),
        jax.ShapeDtypeStruct((nw, B, 16), jnp.float32),
    ]

    @functools.partial(
        pl.kernel, mesh=mesh, out_type=out_type,
        scratch_types=[
            pltpu.VMEM((B, _CHD), jnp.float32),
            pltpu.VMEM((B, _CHD), jnp.float32),
            pltpu.VMEM((B, 16), jnp.float32),
            pltpu.VMEM((B, 16), jnp.int32),
            pltpu.VMEM((B, 16), jnp.float32),
            pltpu.VMEM((B, 16), jnp.float32),
        ],
    )
    def sck(logits_hbm, noise_hbm, om_hbm, oi_hbm, ot_hbm, oc_hbm,
            xbuf, ubuf, mslab, islab, tslab, cslab):
        w = lax.axis_index("s") * 2 + lax.axis_index("c")
        lanes = lax.iota(jnp.int32, 16)
        colbase = vstart + w * segw

        mslab[...] = jnp.full((B, 16), -1.0, jnp.float32)
        islab[...] = jnp.zeros((B, 16), jnp.int32)
        tslab[...] = jnp.ones((B, 16), jnp.float32)
        cslab[...] = jnp.zeros((B, 16), jnp.float32)

        def do_block(start, nvec):
            def row(r, _):
                def vec(j, carry2):
                    acc_m, acc_i, acc_t, acc_c = carry2
                    x = xbuf[r, pl.ds(j * 16, 16)]
                    u = ubuf[r, pl.ds(j * 16, 16)]
                    t = jnp.exp(-x)
                    a = 0.0005 * t + 0.9995
                    onep = 1.0 + t
                    wv = _neg_ln(u)
                    rr = a / (onep * wv)
                    gate = u * onep < a
                    acc_c = acc_c + jnp.where(gate, 1.0, 0.0)
                    better = rr > acc_m
                    pos = (start + j * 16) + lanes
                    acc_m = jnp.where(better, rr, acc_m)
                    acc_i = jnp.where(better, pos, acc_i)
                    acc_t = jnp.where(better, t, acc_t)
                    return acc_m, acc_i, acc_t, acc_c

                carry = (mslab[r], islab[r], tslab[r], cslab[r])
                acc_m, acc_i, acc_t, acc_c = lax.fori_loop(
                    0, nvec, vec, carry, unroll=4)
                mslab[r] = acc_m
                islab[r] = acc_i
                tslab[r] = acc_t
                cslab[r] = acc_c
                return 0

            lax.fori_loop(0, B, row, 0)

        def chunk(c, _):
            start = colbase + c * _CHD
            pltpu.sync_copy(logits_hbm.at[:, pl.ds(start, _CHD)], xbuf)
            pltpu.sync_copy(noise_hbm.at[:, pl.ds(start, _CHD)], ubuf)
            do_block(start, _CHD // 16)
            return 0

        lax.fori_loop(0, nch, chunk, 0)

        pltpu.sync_copy(mslab, om_hbm.at[w])
        pltpu.sync_copy(islab, oi_hbm.at[w])
        pltpu.sync_copy(tslab, ot_hbm.at[w])
        pltpu.sync_copy(cslab, oc_hbm.at[w])

    return sck(logits, noise)


def kernel(logits, noise):
    B, V = logits.shape
    idx_tc, logp_tc, cnt_tc, m_tc = _tc_call(logits, noise, _VSC, V)
    sc_m, sc_i, sc_t, sc_c = _sc_call(logits, noise, 0, _VSC)

    sc_m2 = jnp.swapaxes(sc_m, 0, 1).reshape(B, -1)
    sc_i2 = jnp.swapaxes(sc_i, 0, 1).reshape(B, -1)
    sc_t2 = jnp.swapaxes(sc_t, 0, 1).reshape(B, -1)
    sc_mrow = jnp.max(sc_m2, axis=1)
    hit = sc_m2 == sc_mrow[:, None]
    sc_idx = jnp.min(jnp.where(hit, sc_i2, V), axis=1)
    sc_tat = jnp.max(jnp.where(hit, sc_t2, -1.0), axis=1)
    sc_logp = jnp.log(0.999 / (1.0 + sc_tat) + 0.0005)
    sc_cnt = jnp.sum(sc_c, axis=(0, 2))

    better = sc_mrow >= m_tc[:, 0]
    idx = jnp.where(better, sc_idx, idx_tc[:, 0])
    chosen_logp = jnp.where(better, sc_logp, logp_tc[:, 0])
    gate_count = cnt_tc[:, 0] + sc_cnt
    return (idx, chosen_logp, gate_count)
